# Initial kernel scaffold; baseline (speedup 1.0000x reference)
import jax
import jax.numpy as jnp
from jax.experimental import pallas as pl


def kernel(x, edge_index, W1, b1, W2, b2):
    raise NotImplementedError("write your pallas kernel here")


# speedup vs baseline: 19.1743x; 19.1743x over previous
import functools

import jax
import jax.numpy as jnp
from jax import lax
from jax.experimental import pallas as pl
from jax.experimental.pallas import tpu as pltpu
from jax.experimental.pallas import tpu_sc as plsc

N = 10000
E = 320000
D = 128
NP = 10240
NC = 2
NS = 16
NW = NC * NS
EPW = E // NW
CH = 80
NCHUNK = EPW // CH
RPS = NP // NS

_MESH = plsc.VectorSubcoreMesh(core_axis_name="c", subcore_axis_name="s")


@functools.partial(
    pl.kernel,
    out_type=jax.ShapeDtypeStruct((NC, NP), jnp.float32),
    mesh=_MESH,
    scratch_types=[
        pltpu.VMEM((NCHUNK, CH), jnp.int32),
        pltpu.VMEM((CH,), jnp.float32),
        pltpu.VMEM_SHARED((NP,), jnp.float32),
    ],
)
def _deg_kernel(dst_hbm, zeros_hbm, out_hbm, dst_v, ones_v, acc_sh):
    cid = lax.axis_index("c")
    sid = lax.axis_index("s")
    tile = sid * NC + cid
    pltpu.sync_copy(dst_hbm.at[tile], dst_v)
    for i in range(CH // 16):
        ones_v[pl.ds(i * 16, 16)] = jnp.ones((16,), jnp.float32)
    pltpu.sync_copy(zeros_hbm.at[pl.ds(sid * RPS, RPS)],
                    acc_sh.at[pl.ds(sid * RPS, RPS)])
    plsc.subcore_barrier()

    def body(j, carry):
        pltpu.sync_copy(ones_v, acc_sh.at[dst_v.at[j]], add=True)
        return carry

    lax.fori_loop(0, NCHUNK, body, 0)
    plsc.subcore_barrier()
    pltpu.sync_copy(acc_sh.at[pl.ds(sid * RPS, RPS)],
                    out_hbm.at[cid, pl.ds(sid * RPS, RPS)])


@functools.partial(
    pl.kernel,
    out_type=jax.ShapeDtypeStruct((NC, NP, D), jnp.float32),
    mesh=_MESH,
    scratch_types=[
        pltpu.VMEM((NCHUNK, CH), jnp.int32),
        pltpu.VMEM((NCHUNK, CH), jnp.int32),
        pltpu.VMEM((CH, D), jnp.float32),
        pltpu.VMEM_SHARED((NP, D), jnp.float32),
        pltpu.SemaphoreType.DMA,
    ],
)
def _scatter_kernel(y_hbm, src_hbm, dst_hbm, out_hbm,
                    src_v, dst_v, rows_v, acc_sh, sem):
    cid = lax.axis_index("c")
    sid = lax.axis_index("s")
    tile = sid * NC + cid
    pltpu.sync_copy(src_hbm.at[tile], src_v)
    pltpu.sync_copy(dst_hbm.at[tile], dst_v)
    pltpu.sync_copy(y_hbm.at[pl.ds(sid * RPS, RPS)],
                    acc_sh.at[pl.ds(sid * RPS, RPS)])
    plsc.subcore_barrier()

    def body(j, carry):
        pltpu.async_copy(y_hbm.at[src_v.at[j]], rows_v, sem).wait()
        pltpu.sync_copy(rows_v, acc_sh.at[dst_v.at[j]], add=True)
        return carry

    lax.fori_loop(0, NCHUNK, body, 0)
    plsc.subcore_barrier()
    pltpu.sync_copy(acc_sh.at[pl.ds(sid * RPS, RPS)],
                    out_hbm.at[cid, pl.ds(sid * RPS, RPS)])


BM = 2048


def _mm_body(x_ref, w_ref, o_ref):
    o_ref[...] = jnp.dot(x_ref[...], w_ref[...],
                         preferred_element_type=jnp.float32)


def _matmul(x, w):
    return pl.pallas_call(
        _mm_body,
        grid=(NP // BM,),
        in_specs=[
            pl.BlockSpec((BM, D), lambda i: (i, 0)),
            pl.BlockSpec((D, D), lambda i: (0, 0)),
        ],
        out_specs=pl.BlockSpec((BM, D), lambda i: (i, 0)),
        out_shape=jax.ShapeDtypeStruct((NP, D), jnp.float32),
    )(x, w)


def _scale_body(d0_ref, d1_ref, xw_ref, y_ref, dis_ref):
    dis = lax.rsqrt(d0_ref[...] + d1_ref[...] + 1.0)
    dis_ref[...] = dis
    y_ref[...] = xw_ref[...] * dis


def _scale(d0, d1, xw):
    return pl.pallas_call(
        _scale_body,
        grid=(NP // BM,),
        in_specs=[
            pl.BlockSpec((BM, 1), lambda i: (i, 0)),
            pl.BlockSpec((BM, 1), lambda i: (i, 0)),
            pl.BlockSpec((BM, D), lambda i: (i, 0)),
        ],
        out_specs=[
            pl.BlockSpec((BM, D), lambda i: (i, 0)),
            pl.BlockSpec((BM, 1), lambda i: (i, 0)),
        ],
        out_shape=[
            jax.ShapeDtypeStruct((NP, D), jnp.float32),
            jax.ShapeDtypeStruct((NP, 1), jnp.float32),
        ],
    )(d0, d1, xw)


def _mid_body(p0_ref, p1_ref, y_ref, dis_ref, b_ref, w_ref, o_ref):
    h = dis_ref[...] * (p0_ref[...] + p1_ref[...] - y_ref[...]) + b_ref[...]
    h = jnp.maximum(h, 0.0)
    o_ref[...] = dis_ref[...] * jnp.dot(h, w_ref[...],
                                        preferred_element_type=jnp.float32)


def _mid(p0, p1, y, dis, b, w):
    return pl.pallas_call(
        _mid_body,
        grid=(NP // BM,),
        in_specs=[
            pl.BlockSpec((BM, D), lambda i: (i, 0)),
            pl.BlockSpec((BM, D), lambda i: (i, 0)),
            pl.BlockSpec((BM, D), lambda i: (i, 0)),
            pl.BlockSpec((BM, 1), lambda i: (i, 0)),
            pl.BlockSpec((1, D), lambda i: (0, 0)),
            pl.BlockSpec((D, D), lambda i: (0, 0)),
        ],
        out_specs=pl.BlockSpec((BM, D), lambda i: (i, 0)),
        out_shape=jax.ShapeDtypeStruct((NP, D), jnp.float32),
    )(p0, p1, y, dis, b, w)


def _final_body(p0_ref, p1_ref, y_ref, dis_ref, b_ref, o_ref):
    o_ref[...] = (dis_ref[...] * (p0_ref[...] + p1_ref[...] - y_ref[...])
                  + b_ref[...])


def _final(p0, p1, y, dis, b):
    return pl.pallas_call(
        _final_body,
        grid=(NP // BM,),
        in_specs=[
            pl.BlockSpec((BM, D), lambda i: (i, 0)),
            pl.BlockSpec((BM, D), lambda i: (i, 0)),
            pl.BlockSpec((BM, D), lambda i: (i, 0)),
            pl.BlockSpec((BM, 1), lambda i: (i, 0)),
            pl.BlockSpec((1, D), lambda i: (0, 0)),
        ],
        out_specs=pl.BlockSpec((BM, D), lambda i: (i, 0)),
        out_shape=jax.ShapeDtypeStruct((NP, D), jnp.float32),
    )(p0, p1, y, dis, b)


def kernel(x, edge_index, W1, b1, W2, b2):
    src = edge_index[0].reshape(NW, NCHUNK, CH)
    dst = edge_index[1].reshape(NW, NCHUNK, CH)
    zeros_deg = jnp.zeros((NP,), jnp.float32)
    xp = jnp.pad(x, ((0, NP - N), (0, 0)))

    degp = _deg_kernel(dst, zeros_deg)
    d0 = degp[0].reshape(NP, 1)
    d1 = degp[1].reshape(NP, 1)

    xw1 = _matmul(xp, W1)
    y1, dis = _scale(d0, d1, xw1)

    p1 = _scatter_kernel(y1, src, dst)
    y2 = _mid(p1[0], p1[1], y1, dis, b1.reshape(1, D), W2)

    p2 = _scatter_kernel(y2, src, dst)
    out = _final(p2[0], p2[1], y2, dis, b2.reshape(1, D))
    return out[:N]


# speedup vs baseline: 31.4384x; 1.6396x over previous
import functools

import jax
import jax.numpy as jnp
from jax import lax
from jax.experimental import pallas as pl
from jax.experimental.pallas import tpu as pltpu
from jax.experimental.pallas import tpu_sc as plsc

N = 10000
E = 320000
D = 128
NP = 10240
NC = 2
NS = 16
NW = NC * NS
EPW = E // NW
CH = 80
NCHUNK = EPW // CH
RPS = NP // NS
NBUF = 4
GAHEAD = 2
IAHEAD = 4
IRING = 8

_MESH = plsc.VectorSubcoreMesh(core_axis_name="c", subcore_axis_name="s")


@functools.partial(
    pl.kernel,
    out_type=jax.ShapeDtypeStruct((NC, NP), jnp.float32),
    mesh=_MESH,
    scratch_types=[
        pltpu.VMEM((NCHUNK, CH), jnp.int32),
        pltpu.VMEM((CH,), jnp.float32),
        pltpu.VMEM_SHARED((NP,), jnp.float32),
    ],
)
def _deg_kernel(dst_hbm, zeros_hbm, out_hbm, dst_v, ones_v, acc_sh):
    cid = lax.axis_index("c")
    sid = lax.axis_index("s")
    tile = sid * NC + cid
    pltpu.sync_copy(dst_hbm.at[tile], dst_v)
    for i in range(CH // 16):
        ones_v[pl.ds(i * 16, 16)] = jnp.ones((16,), jnp.float32)
    pltpu.sync_copy(zeros_hbm.at[pl.ds(sid * RPS, RPS)],
                    acc_sh.at[pl.ds(sid * RPS, RPS)])
    plsc.subcore_barrier()

    def body(j, carry):
        pltpu.sync_copy(ones_v, acc_sh.at[dst_v.at[j]], add=True)
        return carry

    lax.fori_loop(0, NCHUNK, body, 0)
    plsc.subcore_barrier()
    pltpu.sync_copy(acc_sh.at[pl.ds(sid * RPS, RPS)],
                    out_hbm.at[cid, pl.ds(sid * RPS, RPS)])


@functools.partial(
    pl.kernel,
    out_type=jax.ShapeDtypeStruct((NC, NP, D), jnp.float32),
    mesh=_MESH,
    scratch_types=[
        pltpu.VMEM((IRING, 2, CH), jnp.int32),
        pltpu.VMEM((NBUF * CH, D), jnp.float32),
        pltpu.VMEM_SHARED((NP, D), jnp.float32),
        pltpu.SemaphoreType.DMA,
        pltpu.SemaphoreType.DMA,
        pltpu.SemaphoreType.DMA,
    ],
)
def _scatter_kernel(y_hbm, ein_hbm, out_hbm,
                    idx_v, rows_v, acc_sh, isem, gsem, ssem):
    cid = lax.axis_index("c")
    sid = lax.axis_index("s")
    tile = sid * NC + cid
    pltpu.sync_copy(y_hbm.at[pl.ds(sid * RPS, RPS)],
                    acc_sh.at[pl.ds(sid * RPS, RPS)])
    plsc.subcore_barrier()

    for k in range(IAHEAD):
        pltpu.async_copy(ein_hbm.at[tile, k], idx_v.at[k], isem)
    for g in range(GAHEAD):
        pltpu.make_async_copy(ein_hbm.at[tile, g], idx_v.at[g], isem).wait()
        pltpu.async_copy(y_hbm.at[idx_v.at[g, 0]],
                         rows_v.at[pl.ds(g * CH, CH)], gsem)

    def body(j, carry):
        @pl.when(j < NCHUNK)
        def _():
            js = lax.rem(j, IRING)
            jb = lax.rem(j, NBUF) * CH
            pltpu.make_async_copy(y_hbm.at[idx_v.at[js, 0]],
                                  rows_v.at[pl.ds(jb, CH)], gsem).wait()
            pltpu.async_copy(rows_v.at[pl.ds(jb, CH)],
                             acc_sh.at[idx_v.at[js, 1]], ssem, add=True)

        @pl.when(j >= GAHEAD)
        def _():
            jj = j - GAHEAD
            js = lax.rem(jj, IRING)
            pltpu.make_async_copy(rows_v.at[pl.ds(lax.rem(jj, NBUF) * CH, CH)],
                                  acc_sh.at[idx_v.at[js, 1]], ssem).wait()

        @pl.when(j + GAHEAD < NCHUNK)
        def _():
            jg = j + GAHEAD
            js = lax.rem(jg, IRING)
            pltpu.make_async_copy(ein_hbm.at[tile, jg], idx_v.at[js],
                                  isem).wait()
            pltpu.async_copy(y_hbm.at[idx_v.at[js, 0]],
                             rows_v.at[pl.ds(lax.rem(jg, NBUF) * CH, CH)],
                             gsem)

        @pl.when(j + IAHEAD < NCHUNK)
        def _():
            ji = j + IAHEAD
            pltpu.async_copy(ein_hbm.at[tile, ji],
                             idx_v.at[lax.rem(ji, IRING)], isem)

        return carry

    lax.fori_loop(0, NCHUNK + GAHEAD, body, 0)
    plsc.subcore_barrier()
    pltpu.sync_copy(acc_sh.at[pl.ds(sid * RPS, RPS)],
                    out_hbm.at[cid, pl.ds(sid * RPS, RPS)])


BM = 2048


def _mm_body(x_ref, w_ref, o_ref):
    o_ref[...] = jnp.dot(x_ref[...], w_ref[...],
                         preferred_element_type=jnp.float32)


def _matmul(x, w):
    return pl.pallas_call(
        _mm_body,
        grid=(NP // BM,),
        in_specs=[
            pl.BlockSpec((BM, D), lambda i: (i, 0)),
            pl.BlockSpec((D, D), lambda i: (0, 0)),
        ],
        out_specs=pl.BlockSpec((BM, D), lambda i: (i, 0)),
        out_shape=jax.ShapeDtypeStruct((NP, D), jnp.float32),
    )(x, w)


def _scale_body(d0_ref, d1_ref, xw_ref, y_ref, dis_ref):
    dis = lax.rsqrt(d0_ref[...] + d1_ref[...] + 1.0)
    dis_ref[...] = dis
    y_ref[...] = xw_ref[...] * dis


def _scale(d0, d1, xw):
    return pl.pallas_call(
        _scale_body,
        grid=(NP // BM,),
        in_specs=[
            pl.BlockSpec((BM, 1), lambda i: (i, 0)),
            pl.BlockSpec((BM, 1), lambda i: (i, 0)),
            pl.BlockSpec((BM, D), lambda i: (i, 0)),
        ],
        out_specs=[
            pl.BlockSpec((BM, D), lambda i: (i, 0)),
            pl.BlockSpec((BM, 1), lambda i: (i, 0)),
        ],
        out_shape=[
            jax.ShapeDtypeStruct((NP, D), jnp.float32),
            jax.ShapeDtypeStruct((NP, 1), jnp.float32),
        ],
    )(d0, d1, xw)


def _mid_body(p0_ref, p1_ref, y_ref, dis_ref, b_ref, w_ref, o_ref):
    h = dis_ref[...] * (p0_ref[...] + p1_ref[...] - y_ref[...]) + b_ref[...]
    h = jnp.maximum(h, 0.0)
    o_ref[...] = dis_ref[...] * jnp.dot(h, w_ref[...],
                                        preferred_element_type=jnp.float32)


def _mid(p0, p1, y, dis, b, w):
    return pl.pallas_call(
        _mid_body,
        grid=(NP // BM,),
        in_specs=[
            pl.BlockSpec((BM, D), lambda i: (i, 0)),
            pl.BlockSpec((BM, D), lambda i: (i, 0)),
            pl.BlockSpec((BM, D), lambda i: (i, 0)),
            pl.BlockSpec((BM, 1), lambda i: (i, 0)),
            pl.BlockSpec((1, D), lambda i: (0, 0)),
            pl.BlockSpec((D, D), lambda i: (0, 0)),
        ],
        out_specs=pl.BlockSpec((BM, D), lambda i: (i, 0)),
        out_shape=jax.ShapeDtypeStruct((NP, D), jnp.float32),
    )(p0, p1, y, dis, b, w)


def _final_body(p0_ref, p1_ref, y_ref, dis_ref, b_ref, o_ref):
    o_ref[...] = (dis_ref[...] * (p0_ref[...] + p1_ref[...] - y_ref[...])
                  + b_ref[...])


def _final(p0, p1, y, dis, b):
    return pl.pallas_call(
        _final_body,
        grid=(NP // BM,),
        in_specs=[
            pl.BlockSpec((BM, D), lambda i: (i, 0)),
            pl.BlockSpec((BM, D), lambda i: (i, 0)),
            pl.BlockSpec((BM, D), lambda i: (i, 0)),
            pl.BlockSpec((BM, 1), lambda i: (i, 0)),
            pl.BlockSpec((1, D), lambda i: (0, 0)),
        ],
        out_specs=pl.BlockSpec((BM, D), lambda i: (i, 0)),
        out_shape=jax.ShapeDtypeStruct((NP, D), jnp.float32),
    )(p0, p1, y, dis, b)


def kernel(x, edge_index, W1, b1, W2, b2):
    dst = edge_index[1].reshape(NW, NCHUNK, CH)
    ein = edge_index.reshape(2, NW, NCHUNK, CH).transpose(1, 2, 0, 3)
    zeros_deg = jnp.zeros((NP,), jnp.float32)
    xp = jnp.pad(x, ((0, NP - N), (0, 0)))

    degp = _deg_kernel(dst, zeros_deg)
    d0 = degp[0].reshape(NP, 1)
    d1 = degp[1].reshape(NP, 1)

    xw1 = _matmul(xp, W1)
    y1, dis = _scale(d0, d1, xw1)

    p1 = _scatter_kernel(y1, ein)
    y2 = _mid(p1[0], p1[1], y1, dis, b1.reshape(1, D), W2)

    p2 = _scatter_kernel(y2, ein)
    out = _final(p2[0], p2[1], y2, dis, b2.reshape(1, D))
    return out[:N]


# speedup vs baseline: 31.8669x; 1.0136x over previous
import functools

import jax
import jax.numpy as jnp
from jax import lax
from jax.experimental import pallas as pl
from jax.experimental.pallas import tpu as pltpu
from jax.experimental.pallas import tpu_sc as plsc

N = 10000
E = 320000
D = 128
NP = 10240
NC = 2
NS = 16
NW = NC * NS
EPW = E // NW
CH = 80
NCHUNK = EPW // CH
RPS = NP // NS
NBUF = 4
GAHEAD = 2
IAHEAD = 4
IRING = 8

_MESH = plsc.VectorSubcoreMesh(core_axis_name="c", subcore_axis_name="s")


@functools.partial(
    pl.kernel,
    out_type=jax.ShapeDtypeStruct((NC, NP), jnp.float32),
    mesh=_MESH,
    scratch_types=[
        pltpu.VMEM((NCHUNK, CH), jnp.int32),
        pltpu.VMEM((CH,), jnp.float32),
        pltpu.VMEM_SHARED((NP,), jnp.float32),
        pltpu.SemaphoreType.DMA,
    ],
)
def _deg_kernel(dst_hbm, zeros_hbm, out_hbm, dst_v, ones_v, acc_sh, sem):
    cid = lax.axis_index("c")
    sid = lax.axis_index("s")
    tile = sid * NC + cid
    pltpu.sync_copy(dst_hbm.at[tile], dst_v)
    for i in range(CH // 16):
        ones_v[pl.ds(i * 16, 16)] = jnp.ones((16,), jnp.float32)
    pltpu.sync_copy(zeros_hbm.at[pl.ds(sid * RPS, RPS)],
                    acc_sh.at[pl.ds(sid * RPS, RPS)])
    plsc.subcore_barrier()

    def body(j, carry):
        @pl.when(j < NCHUNK)
        def _():
            pltpu.async_copy(ones_v, acc_sh.at[dst_v.at[j]], sem, add=True)

        @pl.when(j >= 4)
        def _():
            pltpu.make_async_copy(ones_v, acc_sh.at[dst_v.at[j - 4]],
                                  sem).wait()

        return carry

    lax.fori_loop(0, NCHUNK + 4, body, 0)
    plsc.subcore_barrier()
    pltpu.sync_copy(acc_sh.at[pl.ds(sid * RPS, RPS)],
                    out_hbm.at[cid, pl.ds(sid * RPS, RPS)])


@functools.partial(
    pl.kernel,
    out_type=jax.ShapeDtypeStruct((NC, NP, D), jnp.float32),
    mesh=_MESH,
    scratch_types=[
        pltpu.VMEM((IRING, 2, CH), jnp.int32),
        pltpu.VMEM((NBUF * CH, D), jnp.float32),
        pltpu.VMEM_SHARED((NP, D), jnp.float32),
        pltpu.SemaphoreType.DMA,
        pltpu.SemaphoreType.DMA,
        pltpu.SemaphoreType.DMA,
    ],
)
def _scatter_kernel(y_hbm, ein_hbm, out_hbm,
                    idx_v, rows_v, acc_sh, isem, gsem, ssem):
    cid = lax.axis_index("c")
    sid = lax.axis_index("s")
    tile = sid * NC + cid
    pltpu.sync_copy(y_hbm.at[pl.ds(sid * RPS, RPS)],
                    acc_sh.at[pl.ds(sid * RPS, RPS)])
    plsc.subcore_barrier()

    for k in range(IAHEAD):
        pltpu.async_copy(ein_hbm.at[tile, k], idx_v.at[k], isem)
    for g in range(GAHEAD):
        pltpu.make_async_copy(ein_hbm.at[tile, g], idx_v.at[g], isem).wait()
        pltpu.async_copy(y_hbm.at[idx_v.at[g, 0]],
                         rows_v.at[pl.ds(g * CH, CH)], gsem)

    def body(j, carry):
        @pl.when(j < NCHUNK)
        def _():
            js = lax.rem(j, IRING)
            jb = lax.rem(j, NBUF) * CH
            pltpu.make_async_copy(y_hbm.at[idx_v.at[js, 0]],
                                  rows_v.at[pl.ds(jb, CH)], gsem).wait()
            pltpu.async_copy(rows_v.at[pl.ds(jb, CH)],
                             acc_sh.at[idx_v.at[js, 1]], ssem, add=True)

        @pl.when(j >= GAHEAD)
        def _():
            jj = j - GAHEAD
            js = lax.rem(jj, IRING)
            pltpu.make_async_copy(rows_v.at[pl.ds(lax.rem(jj, NBUF) * CH, CH)],
                                  acc_sh.at[idx_v.at[js, 1]], ssem).wait()

        @pl.when(j + GAHEAD < NCHUNK)
        def _():
            jg = j + GAHEAD
            js = lax.rem(jg, IRING)
            pltpu.make_async_copy(ein_hbm.at[tile, jg], idx_v.at[js],
                                  isem).wait()
            pltpu.async_copy(y_hbm.at[idx_v.at[js, 0]],
                             rows_v.at[pl.ds(lax.rem(jg, NBUF) * CH, CH)],
                             gsem)

        @pl.when(j + IAHEAD < NCHUNK)
        def _():
            ji = j + IAHEAD
            pltpu.async_copy(ein_hbm.at[tile, ji],
                             idx_v.at[lax.rem(ji, IRING)], isem)

        return carry

    lax.fori_loop(0, NCHUNK + GAHEAD, body, 0)
    plsc.subcore_barrier()
    pltpu.sync_copy(acc_sh.at[pl.ds(sid * RPS, RPS)],
                    out_hbm.at[cid, pl.ds(sid * RPS, RPS)])


BM = 2048


def _mm_body(x_ref, w_ref, o_ref):
    o_ref[...] = jnp.dot(x_ref[...], w_ref[...],
                         preferred_element_type=jnp.float32)


def _matmul(x, w):
    return pl.pallas_call(
        _mm_body,
        grid=(NP // BM,),
        in_specs=[
            pl.BlockSpec((BM, D), lambda i: (i, 0)),
            pl.BlockSpec((D, D), lambda i: (0, 0)),
        ],
        out_specs=pl.BlockSpec((BM, D), lambda i: (i, 0)),
        out_shape=jax.ShapeDtypeStruct((NP, D), jnp.float32),
    )(x, w)


def _scale_body(d0_ref, d1_ref, xw_ref, y_ref, dis_ref):
    dis = lax.rsqrt(d0_ref[...] + d1_ref[...] + 1.0)
    dis_ref[...] = dis
    y_ref[...] = xw_ref[...] * dis


def _scale(d0, d1, xw):
    return pl.pallas_call(
        _scale_body,
        grid=(NP // BM,),
        in_specs=[
            pl.BlockSpec((BM, 1), lambda i: (i, 0)),
            pl.BlockSpec((BM, 1), lambda i: (i, 0)),
            pl.BlockSpec((BM, D), lambda i: (i, 0)),
        ],
        out_specs=[
            pl.BlockSpec((BM, D), lambda i: (i, 0)),
            pl.BlockSpec((BM, 1), lambda i: (i, 0)),
        ],
        out_shape=[
            jax.ShapeDtypeStruct((NP, D), jnp.float32),
            jax.ShapeDtypeStruct((NP, 1), jnp.float32),
        ],
    )(d0, d1, xw)


def _mid_body(p0_ref, p1_ref, y_ref, dis_ref, b_ref, w_ref, o_ref):
    h = dis_ref[...] * (p0_ref[...] + p1_ref[...] - y_ref[...]) + b_ref[...]
    h = jnp.maximum(h, 0.0)
    o_ref[...] = dis_ref[...] * jnp.dot(h, w_ref[...],
                                        preferred_element_type=jnp.float32)


def _mid(p0, p1, y, dis, b, w):
    return pl.pallas_call(
        _mid_body,
        grid=(NP // BM,),
        in_specs=[
            pl.BlockSpec((BM, D), lambda i: (i, 0)),
            pl.BlockSpec((BM, D), lambda i: (i, 0)),
            pl.BlockSpec((BM, D), lambda i: (i, 0)),
            pl.BlockSpec((BM, 1), lambda i: (i, 0)),
            pl.BlockSpec((1, D), lambda i: (0, 0)),
            pl.BlockSpec((D, D), lambda i: (0, 0)),
        ],
        out_specs=pl.BlockSpec((BM, D), lambda i: (i, 0)),
        out_shape=jax.ShapeDtypeStruct((NP, D), jnp.float32),
    )(p0, p1, y, dis, b, w)


def _final_body(p0_ref, p1_ref, y_ref, dis_ref, b_ref, o_ref):
    o_ref[...] = (dis_ref[...] * (p0_ref[...] + p1_ref[...] - y_ref[...])
                  + b_ref[...])


BF = 2000


def _final(p0, p1, y, dis, b):
    return pl.pallas_call(
        _final_body,
        grid=(N // BF,),
        in_specs=[
            pl.BlockSpec((BF, D), lambda i: (i, 0)),
            pl.BlockSpec((BF, D), lambda i: (i, 0)),
            pl.BlockSpec((BF, D), lambda i: (i, 0)),
            pl.BlockSpec((BF, 1), lambda i: (i, 0)),
            pl.BlockSpec((1, D), lambda i: (0, 0)),
        ],
        out_specs=pl.BlockSpec((BF, D), lambda i: (i, 0)),
        out_shape=jax.ShapeDtypeStruct((N, D), jnp.float32),
    )(p0, p1, y, dis, b)


def kernel(x, edge_index, W1, b1, W2, b2):
    dst = edge_index[1].reshape(NW, NCHUNK, CH)
    ein = edge_index.reshape(2, NW, NCHUNK, CH).transpose(1, 2, 0, 3)
    zeros_deg = jnp.zeros((NP,), jnp.float32)
    xp = jnp.pad(x, ((0, NP - N), (0, 0)))

    degp = _deg_kernel(dst, zeros_deg)
    d0 = degp[0].reshape(NP, 1)
    d1 = degp[1].reshape(NP, 1)

    xw1 = _matmul(xp, W1)
    y1, dis = _scale(d0, d1, xw1)

    p1 = _scatter_kernel(y1, ein)
    y2 = _mid(p1[0], p1[1], y1, dis, b1.reshape(1, D), W2)

    p2 = _scatter_kernel(y2, ein)
    return _final(p2[0], p2[1], y2, dis, b2.reshape(1, D))


# speedup vs baseline: 36.4552x; 1.1440x over previous
import functools

import jax
import jax.numpy as jnp
from jax import lax
from jax.experimental import pallas as pl
from jax.experimental.pallas import tpu as pltpu
from jax.experimental.pallas import tpu_sc as plsc

N = 10000
E = 320000
D = 128
NP = 10240
NC = 2
NS = 16
NW = NC * NS
EPW = E // NW
CH = 80
NCHUNK = EPW // CH
RPS = NP // NS
NBUF = 4
GAHEAD = 2
IAHEAD = 4
IRING = 8

_MESH = plsc.VectorSubcoreMesh(core_axis_name="c", subcore_axis_name="s")


@functools.partial(
    pl.kernel,
    out_type=jax.ShapeDtypeStruct((NC, NP), jnp.float32),
    mesh=_MESH,
    scratch_types=[
        pltpu.VMEM((NCHUNK, CH), jnp.int32),
        pltpu.VMEM((CH,), jnp.float32),
        pltpu.VMEM_SHARED((NP,), jnp.float32),
        pltpu.SemaphoreType.DMA,
    ],
)
def _deg_kernel(dst_hbm, zeros_hbm, out_hbm, dst_v, ones_v, acc_sh, sem):
    cid = lax.axis_index("c")
    sid = lax.axis_index("s")
    tile = sid * NC + cid
    pltpu.sync_copy(dst_hbm.at[tile], dst_v)
    for i in range(CH // 16):
        ones_v[pl.ds(i * 16, 16)] = jnp.ones((16,), jnp.float32)
    pltpu.sync_copy(zeros_hbm.at[pl.ds(sid * RPS, RPS)],
                    acc_sh.at[pl.ds(sid * RPS, RPS)])
    plsc.subcore_barrier()

    def body(j, carry):
        @pl.when(j < NCHUNK)
        def _():
            pltpu.async_copy(ones_v, acc_sh.at[dst_v.at[j]], sem, add=True)

        @pl.when(j >= 4)
        def _():
            pltpu.make_async_copy(ones_v, acc_sh.at[dst_v.at[j - 4]],
                                  sem).wait()

        return carry

    lax.fori_loop(0, NCHUNK + 4, body, 0)
    plsc.subcore_barrier()
    pltpu.sync_copy(acc_sh.at[pl.ds(sid * RPS, RPS)],
                    out_hbm.at[cid, pl.ds(sid * RPS, RPS)])


@functools.partial(
    pl.kernel,
    out_type=jax.ShapeDtypeStruct((NC, NP, D), jnp.float32),
    mesh=_MESH,
    scratch_types=[
        pltpu.VMEM((IRING, 2, CH), jnp.int32),
        pltpu.VMEM((NBUF * CH, D), jnp.float32),
        pltpu.VMEM_SHARED((NP, D), jnp.float32),
        pltpu.SemaphoreType.DMA,
        pltpu.SemaphoreType.DMA,
        pltpu.SemaphoreType.DMA,
    ],
)
def _scatter_kernel(y_hbm, ein_hbm, out_hbm,
                    idx_v, rows_v, acc_sh, isem, gsem, ssem):
    cid = lax.axis_index("c")
    sid = lax.axis_index("s")
    tile = sid * NC + cid
    pltpu.sync_copy(y_hbm.at[pl.ds(sid * RPS, RPS)],
                    acc_sh.at[pl.ds(sid * RPS, RPS)])
    plsc.subcore_barrier()

    for k in range(IAHEAD):
        pltpu.async_copy(ein_hbm.at[0, tile, k], idx_v.at[k, 0], isem)
        pltpu.async_copy(ein_hbm.at[1, tile, k], idx_v.at[k, 1], isem)
    for g in range(GAHEAD):
        pltpu.make_async_copy(ein_hbm.at[0, tile, g], idx_v.at[g, 0],
                              isem).wait()
        pltpu.make_async_copy(ein_hbm.at[1, tile, g], idx_v.at[g, 1],
                              isem).wait()
        pltpu.async_copy(y_hbm.at[idx_v.at[g, 0]],
                         rows_v.at[pl.ds(g * CH, CH)], gsem)

    def body(j, carry):
        @pl.when(j < NCHUNK)
        def _():
            js = lax.rem(j, IRING)
            jb = lax.rem(j, NBUF) * CH
            pltpu.make_async_copy(y_hbm.at[idx_v.at[js, 0]],
                                  rows_v.at[pl.ds(jb, CH)], gsem).wait()
            pltpu.async_copy(rows_v.at[pl.ds(jb, CH)],
                             acc_sh.at[idx_v.at[js, 1]], ssem, add=True)

        @pl.when(j >= GAHEAD)
        def _():
            jj = j - GAHEAD
            js = lax.rem(jj, IRING)
            pltpu.make_async_copy(rows_v.at[pl.ds(lax.rem(jj, NBUF) * CH, CH)],
                                  acc_sh.at[idx_v.at[js, 1]], ssem).wait()

        @pl.when(j + GAHEAD < NCHUNK)
        def _():
            jg = j + GAHEAD
            js = lax.rem(jg, IRING)
            pltpu.make_async_copy(ein_hbm.at[0, tile, jg], idx_v.at[js, 0],
                                  isem).wait()
            pltpu.make_async_copy(ein_hbm.at[1, tile, jg], idx_v.at[js, 1],
                                  isem).wait()
            pltpu.async_copy(y_hbm.at[idx_v.at[js, 0]],
                             rows_v.at[pl.ds(lax.rem(jg, NBUF) * CH, CH)],
                             gsem)

        @pl.when(j + IAHEAD < NCHUNK)
        def _():
            ji = j + IAHEAD
            jr = lax.rem(ji, IRING)
            pltpu.async_copy(ein_hbm.at[0, tile, ji], idx_v.at[jr, 0], isem)
            pltpu.async_copy(ein_hbm.at[1, tile, ji], idx_v.at[jr, 1], isem)

        return carry

    lax.fori_loop(0, NCHUNK + GAHEAD, body, 0)
    plsc.subcore_barrier()
    pltpu.sync_copy(acc_sh.at[pl.ds(sid * RPS, RPS)],
                    out_hbm.at[cid, pl.ds(sid * RPS, RPS)])


BM = 2048


def _mm_body(x_ref, w_ref, o_ref):
    o_ref[...] = jnp.dot(x_ref[...], w_ref[...],
                         preferred_element_type=jnp.float32)


def _matmul(x, w):
    return pl.pallas_call(
        _mm_body,
        grid=(NP // BM,),
        in_specs=[
            pl.BlockSpec((BM, D), lambda i: (i, 0)),
            pl.BlockSpec((D, D), lambda i: (0, 0)),
        ],
        out_specs=pl.BlockSpec((BM, D), lambda i: (i, 0)),
        out_shape=jax.ShapeDtypeStruct((NP, D), jnp.float32),
    )(x, w)


def _scale_body(d_ref, xw_ref, y_ref, dis_ref):
    dis = lax.rsqrt(d_ref[...] + 1.0)
    dis_ref[...] = dis
    y_ref[...] = xw_ref[...] * dis


def _scale(d, xw):
    return pl.pallas_call(
        _scale_body,
        grid=(NP // BM,),
        in_specs=[
            pl.BlockSpec((BM, 1), lambda i: (i, 0)),
            pl.BlockSpec((BM, D), lambda i: (i, 0)),
        ],
        out_specs=[
            pl.BlockSpec((BM, D), lambda i: (i, 0)),
            pl.BlockSpec((BM, 1), lambda i: (i, 0)),
        ],
        out_shape=[
            jax.ShapeDtypeStruct((NP, D), jnp.float32),
            jax.ShapeDtypeStruct((NP, 1), jnp.float32),
        ],
    )(d, xw)


def _mid_body(p0_ref, p1_ref, y_ref, dis_ref, b_ref, w_ref, o_ref):
    h = (dis_ref[...] * (p0_ref[0] + p1_ref[0] - y_ref[...])
         + b_ref[...])
    h = jnp.maximum(h, 0.0)
    o_ref[...] = dis_ref[...] * jnp.dot(h, w_ref[...],
                                        preferred_element_type=jnp.float32)


def _mid(p, y, dis, b, w):
    return pl.pallas_call(
        _mid_body,
        grid=(NP // BM,),
        in_specs=[
            pl.BlockSpec((1, BM, D), lambda i: (0, i, 0)),
            pl.BlockSpec((1, BM, D), lambda i: (1, i, 0)),
            pl.BlockSpec((BM, D), lambda i: (i, 0)),
            pl.BlockSpec((BM, 1), lambda i: (i, 0)),
            pl.BlockSpec((1, D), lambda i: (0, 0)),
            pl.BlockSpec((D, D), lambda i: (0, 0)),
        ],
        out_specs=pl.BlockSpec((BM, D), lambda i: (i, 0)),
        out_shape=jax.ShapeDtypeStruct((NP, D), jnp.float32),
    )(p, p, y, dis, b, w)


def _final_body(p0_ref, p1_ref, y_ref, dis_ref, b_ref, o_ref):
    o_ref[...] = (dis_ref[...] * (p0_ref[0] + p1_ref[0] - y_ref[...])
                  + b_ref[...])


BF = 2000


def _final(p, y, dis, b):
    return pl.pallas_call(
        _final_body,
        grid=(N // BF,),
        in_specs=[
            pl.BlockSpec((1, BF, D), lambda i: (0, i, 0)),
            pl.BlockSpec((1, BF, D), lambda i: (1, i, 0)),
            pl.BlockSpec((BF, D), lambda i: (i, 0)),
            pl.BlockSpec((BF, 1), lambda i: (i, 0)),
            pl.BlockSpec((1, D), lambda i: (0, 0)),
        ],
        out_specs=pl.BlockSpec((BF, D), lambda i: (i, 0)),
        out_shape=jax.ShapeDtypeStruct((N, D), jnp.float32),
    )(p, p, y, dis, b)


def kernel(x, edge_index, W1, b1, W2, b2):
    ein = edge_index.reshape(2, NW, NCHUNK, CH)
    dst = ein[1]
    zeros_deg = jnp.zeros((NP,), jnp.float32)

    degp = _deg_kernel(dst, zeros_deg)
    d = (degp[0] + degp[1]).reshape(NP, 1)

    xw1 = _matmul(x, W1)
    y1, dis = _scale(d, xw1)

    p1 = _scatter_kernel(y1, ein)
    y2 = _mid(p1, y1, dis, b1.reshape(1, D), W2)

    p2 = _scatter_kernel(y2, ein)
    return _final(p2, y2, dis, b2.reshape(1, D))


# speedup vs baseline: 37.3477x; 1.0245x over previous
import functools

import jax
import jax.numpy as jnp
from jax import lax
from jax.experimental import pallas as pl
from jax.experimental.pallas import tpu as pltpu
from jax.experimental.pallas import tpu_sc as plsc

N = 10000
E = 320000
D = 128
NP = 10240
NC = 2
NS = 16
NW = NC * NS
EPW = E // NW
CH = 80
NCHUNK = EPW // CH
RPS = NP // NS
NBUF = 4
GAHEAD = 2
IAHEAD = 4
IRING = 8

_MESH = plsc.VectorSubcoreMesh(core_axis_name="c", subcore_axis_name="s")


@functools.partial(
    pl.kernel,
    out_type=jax.ShapeDtypeStruct((NC, NP), jnp.float32),
    mesh=_MESH,
    scratch_types=[
        pltpu.VMEM((NCHUNK, CH), jnp.int32),
        pltpu.VMEM((CH,), jnp.float32),
        pltpu.VMEM_SHARED((NP,), jnp.float32),
        pltpu.SemaphoreType.DMA,
    ],
)
def _deg_kernel(dst_hbm, zeros_hbm, out_hbm, dst_v, ones_v, acc_sh, sem):
    cid = lax.axis_index("c")
    sid = lax.axis_index("s")
    tile = sid * NC + cid
    pltpu.sync_copy(dst_hbm.at[tile], dst_v)
    for i in range(CH // 16):
        ones_v[pl.ds(i * 16, 16)] = jnp.ones((16,), jnp.float32)
    pltpu.sync_copy(zeros_hbm.at[pl.ds(sid * RPS, RPS)],
                    acc_sh.at[pl.ds(sid * RPS, RPS)])
    plsc.subcore_barrier()

    def body(j, carry):
        @pl.when(j < NCHUNK)
        def _():
            pltpu.async_copy(ones_v, acc_sh.at[dst_v.at[j]], sem, add=True)

        @pl.when(j >= 4)
        def _():
            pltpu.make_async_copy(ones_v, acc_sh.at[dst_v.at[j - 4]],
                                  sem).wait()

        return carry

    lax.fori_loop(0, NCHUNK + 4, body, 0)
    plsc.subcore_barrier()
    pltpu.sync_copy(acc_sh.at[pl.ds(sid * RPS, RPS)],
                    out_hbm.at[cid, pl.ds(sid * RPS, RPS)])


@functools.partial(
    pl.kernel,
    out_type=jax.ShapeDtypeStruct((NC, NP, D), jnp.float32),
    mesh=_MESH,
    scratch_types=[
        pltpu.VMEM((IRING, 2, CH), jnp.int32),
        pltpu.VMEM((NBUF * CH, D), jnp.float32),
        pltpu.VMEM_SHARED((NP, D), jnp.float32),
        pltpu.SemaphoreType.DMA,
        pltpu.SemaphoreType.DMA,
        pltpu.SemaphoreType.DMA,
    ],
)
def _scatter_kernel(y_hbm, ein_hbm, out_hbm,
                    idx_v, rows_v, acc_sh, isem, gsem, ssem):
    cid = lax.axis_index("c")
    sid = lax.axis_index("s")
    tile = sid * NC + cid
    pltpu.sync_copy(y_hbm.at[pl.ds(sid * RPS, RPS)],
                    acc_sh.at[pl.ds(sid * RPS, RPS)])
    plsc.subcore_barrier()

    def _start_idx(ji):
        jr = lax.rem(ji, IRING)
        pltpu.async_copy(ein_hbm.at[0, tile, ji], idx_v.at[jr, 0], isem)
        pltpu.async_copy(ein_hbm.at[1, tile, ji], idx_v.at[jr, 1], isem)

    def _wait_idx_start_gather(jg):
        js = lax.rem(jg, IRING)
        pltpu.make_async_copy(ein_hbm.at[0, tile, jg], idx_v.at[js, 0],
                              isem).wait()
        pltpu.make_async_copy(ein_hbm.at[1, tile, jg], idx_v.at[js, 1],
                              isem).wait()
        pltpu.async_copy(y_hbm.at[idx_v.at[js, 0]],
                         rows_v.at[pl.ds(lax.rem(jg, NBUF) * CH, CH)], gsem)

    def _wait_gather_start_scatter(j):
        js = lax.rem(j, IRING)
        jb = lax.rem(j, NBUF) * CH
        pltpu.make_async_copy(y_hbm.at[idx_v.at[js, 0]],
                              rows_v.at[pl.ds(jb, CH)], gsem).wait()
        pltpu.async_copy(rows_v.at[pl.ds(jb, CH)],
                         acc_sh.at[idx_v.at[js, 1]], ssem, add=True)

    def _wait_scatter(jj):
        js = lax.rem(jj, IRING)
        pltpu.make_async_copy(rows_v.at[pl.ds(lax.rem(jj, NBUF) * CH, CH)],
                              acc_sh.at[idx_v.at[js, 1]], ssem).wait()

    for k in range(IAHEAD):
        _start_idx(k)
    for g in range(GAHEAD):
        _wait_idx_start_gather(g)

    def body(j, carry):
        @pl.when(j < NCHUNK)
        def _():
            _wait_gather_start_scatter(j)

        @pl.when(j >= GAHEAD)
        def _():
            _wait_scatter(j - GAHEAD)

        @pl.when(j + GAHEAD < NCHUNK)
        def _():
            _wait_idx_start_gather(j + GAHEAD)

        @pl.when(j + IAHEAD < NCHUNK)
        def _():
            _start_idx(j + IAHEAD)

        return carry

    lax.fori_loop(0, NCHUNK + GAHEAD, body, 0)
    plsc.subcore_barrier()
    pltpu.sync_copy(acc_sh.at[pl.ds(sid * RPS, RPS)],
                    out_hbm.at[cid, pl.ds(sid * RPS, RPS)])


BM = 2048


def _mmscale_body(d_ref, x_ref, w_ref, y_ref, dis_ref):
    dis_r = lax.rsqrt(d_ref[...] + 1.0)
    dis_ref[...] = dis_r
    xw = jnp.dot(x_ref[...], w_ref[...], preferred_element_type=jnp.float32)
    y_ref[...] = xw * lax.transpose(dis_r, (1, 0))


def _mmscale(d, x, w):
    return pl.pallas_call(
        _mmscale_body,
        grid=(NP // BM,),
        in_specs=[
            pl.BlockSpec((1, BM), lambda i: (0, i)),
            pl.BlockSpec((BM, D), lambda i: (i, 0)),
            pl.BlockSpec((D, D), lambda i: (0, 0)),
        ],
        out_specs=[
            pl.BlockSpec((BM, D), lambda i: (i, 0)),
            pl.BlockSpec((1, BM), lambda i: (0, i)),
        ],
        out_shape=[
            jax.ShapeDtypeStruct((NP, D), jnp.float32),
            jax.ShapeDtypeStruct((1, NP), jnp.float32),
        ],
    )(d, x, w)


def _mid_body(p0_ref, p1_ref, y_ref, dis_ref, b_ref, w_ref, o_ref):
    dis = lax.transpose(dis_ref[...], (1, 0))
    h = dis * (p0_ref[0] + p1_ref[0] - y_ref[...]) + b_ref[...]
    h = jnp.maximum(h, 0.0)
    o_ref[...] = dis * jnp.dot(h, w_ref[...],
                               preferred_element_type=jnp.float32)


def _mid(p, y, dis, b, w):
    return pl.pallas_call(
        _mid_body,
        grid=(NP // BM,),
        in_specs=[
            pl.BlockSpec((1, BM, D), lambda i: (0, i, 0)),
            pl.BlockSpec((1, BM, D), lambda i: (1, i, 0)),
            pl.BlockSpec((BM, D), lambda i: (i, 0)),
            pl.BlockSpec((1, BM), lambda i: (0, i)),
            pl.BlockSpec((1, D), lambda i: (0, 0)),
            pl.BlockSpec((D, D), lambda i: (0, 0)),
        ],
        out_specs=pl.BlockSpec((BM, D), lambda i: (i, 0)),
        out_shape=jax.ShapeDtypeStruct((NP, D), jnp.float32),
    )(p, p, y, dis, b, w)


def _final_body(p0_ref, p1_ref, y_ref, dis_ref, b_ref, o_ref):
    dis = lax.transpose(dis_ref[...], (1, 0))
    o_ref[...] = dis * (p0_ref[0] + p1_ref[0] - y_ref[...]) + b_ref[...]


BF = 2048


def _final(p, y, dis, b):
    return pl.pallas_call(
        _final_body,
        grid=(NP // BF,),
        in_specs=[
            pl.BlockSpec((1, BF, D), lambda i: (0, i, 0)),
            pl.BlockSpec((1, BF, D), lambda i: (1, i, 0)),
            pl.BlockSpec((BF, D), lambda i: (i, 0)),
            pl.BlockSpec((1, BF), lambda i: (0, i)),
            pl.BlockSpec((1, D), lambda i: (0, 0)),
        ],
        out_specs=pl.BlockSpec((BF, D), lambda i: (i, 0)),
        out_shape=jax.ShapeDtypeStruct((N, D), jnp.float32),
    )(p, p, y, dis, b)


def kernel(x, edge_index, W1, b1, W2, b2):
    ein = edge_index.reshape(2, NW, NCHUNK, CH)
    dst = ein[1]
    zeros_deg = jnp.zeros((NP,), jnp.float32)

    degp = _deg_kernel(dst, zeros_deg)
    d = (degp[0] + degp[1]).reshape(1, NP)

    y1, dis = _mmscale(d, x, W1)

    p1 = _scatter_kernel(y1, ein)
    y2 = _mid(p1, y1, dis, b1.reshape(1, D), W2)

    p2 = _scatter_kernel(y2, ein)
    return _final(p2, y2, dis, b2.reshape(1, D))
